# baseline (speedup 1.0000x reference)
import functools

import jax
import jax.numpy as jnp
from jax import lax
from jax.experimental import pallas as pl
from jax.experimental.pallas import tpu as pltpu
from jax.experimental.pallas import tpu_sc as plsc

_M = 64
_K = 16
_LANES = 16
_VPR = _M // _LANES
_R = 256
_CS = _R * _M


def _lane_shuffle(v, perm):
    dnums = lax.GatherDimensionNumbers(
        offset_dims=(), collapsed_slice_dims=(0,), start_index_map=(0,)
    )
    return lax.gather(
        v,
        perm[:, None],
        dimension_numbers=dnums,
        slice_sizes=(1,),
        mode=lax.GatherScatterMode.PROMISE_IN_BOUNDS,
    )


def _lane_all_sum(v, perms):
    for perm in perms:
        v = v + _lane_shuffle(v, perm)
    return v


def _do_row(lbuf, gbuf, obuf, off, perms, tiny):
    x = [
        lbuf[pl.ds(off + j * _LANES, _LANES)] + gbuf[pl.ds(off + j * _LANES, _LANES)]
        for j in range(_VPR)
    ]
    e = [jnp.exp(xj) for xj in x]
    s = e[0] + e[1] + e[2] + e[3]
    r = 1.0 / _lane_all_sum(s, perms)
    p = [ej * r for ej in e]
    for j in range(_VPR):
        obuf[pl.ds(off + j * _LANES, _LANES)] = p[j]
    for _ in range(_K - 1):
        w = [pj * jnp.maximum(1.0 - pj, tiny) for pj in p]
        s = (w[0] + w[1]) + (w[2] + w[3])
        r = 1.0 / _lane_all_sum(s, perms)
        p = [wj * r for wj in w]
        for j in range(_VPR):
            plsc.addupdate(obuf.at[pl.ds(off + j * _LANES, _LANES)], p[j])


def _sc_kernel_body(l_hbm, g_hbm, o_hbm, lbufs, gbufs, obufs, lsems, gsems, osems):
    info = plsc.get_sparse_core_info()
    nc = info.num_cores
    nw = nc * info.num_subcores
    wid = lax.axis_index("s") * nc + lax.axis_index("c")

    n_total = l_hbm.shape[0] // _M
    rows_per_w = n_total // nw
    n_chunks = rows_per_w // _R
    w_base = wid * rows_per_w * _M
    tiny = jnp.float32(jnp.finfo(jnp.float32).tiny)
    lane = lax.iota(jnp.int32, _LANES)
    perms = [lane ^ sh for sh in (1, 2, 4, 8)]

    def start_in(ci, b):
        base = w_base + ci * _CS
        pltpu.make_async_copy(l_hbm.at[pl.ds(base, _CS)], lbufs[b], lsems[b]).start()
        pltpu.make_async_copy(g_hbm.at[pl.ds(base, _CS)], gbufs[b], gsems[b]).start()

    start_in(0, 0)
    start_in(1, 1)

    def pair_body(i, _):
        for b in range(2):
            ci = 2 * i + b
            base = w_base + ci * _CS
            pltpu.make_async_copy(
                l_hbm.at[pl.ds(base, _CS)], lbufs[b], lsems[b]
            ).wait()
            pltpu.make_async_copy(
                g_hbm.at[pl.ds(base, _CS)], gbufs[b], gsems[b]
            ).wait()

            @pl.when(ci >= 2)
            def _():
                pltpu.make_async_copy(
                    obufs[b], o_hbm.at[pl.ds(base - 2 * _CS, _CS)], osems[b]
                ).wait()

            def row_body(ri, _):
                _do_row(lbufs[b], gbufs[b], obufs[b], ri * _M, perms, tiny)
                return 0

            lax.fori_loop(0, _R, row_body, 0, unroll=2)

            pltpu.make_async_copy(
                obufs[b], o_hbm.at[pl.ds(base, _CS)], osems[b]
            ).start()

            @pl.when(ci + 2 < n_chunks)
            def _():
                start_in(ci + 2, b)

        return 0

    lax.fori_loop(0, n_chunks // 2, pair_body, 0)

    for b in range(2):
        ci = n_chunks - 2 + b
        pltpu.make_async_copy(
            obufs[b], o_hbm.at[pl.ds(w_base + ci * _CS, _CS)], osems[b]
        ).wait()


def kernel(logits, gumbel):
    n, m = logits.shape
    mesh = plsc.VectorSubcoreMesh(core_axis_name="c", subcore_axis_name="s")
    buf = lambda: pltpu.VMEM((_CS,), jnp.float32)
    run = functools.partial(
        pl.kernel,
        mesh=mesh,
        out_type=jax.ShapeDtypeStruct((n * m,), jnp.float32),
        scratch_types=[
            [buf(), buf()],
            [buf(), buf()],
            [buf(), buf()],
            [pltpu.SemaphoreType.DMA, pltpu.SemaphoreType.DMA],
            [pltpu.SemaphoreType.DMA, pltpu.SemaphoreType.DMA],
            [pltpu.SemaphoreType.DMA, pltpu.SemaphoreType.DMA],
        ],
    )(_sc_kernel_body)
    out = run(logits.reshape(-1), gumbel.reshape(-1))
    return out.reshape(n, m)


# speedup vs baseline: 2.3665x; 2.3665x over previous
import functools

import jax
import jax.numpy as jnp
from jax import lax
from jax.experimental import pallas as pl
from jax.experimental.pallas import tpu as pltpu
from jax.experimental.pallas import tpu_sc as plsc

_M = 64
_K = 16
_LANES = 16
_VPR = _M // _LANES
_R = 256
_CS = _R * _M


def _lane_shuffle(v, perm):
    dnums = lax.GatherDimensionNumbers(
        offset_dims=(), collapsed_slice_dims=(0,), start_index_map=(0,)
    )
    return lax.gather(
        v,
        perm[:, None],
        dimension_numbers=dnums,
        slice_sizes=(1,),
        mode=lax.GatherScatterMode.PROMISE_IN_BOUNDS,
    )


def _lane_all_sum(v, perms):
    for perm in perms:
        v = v + _lane_shuffle(v, perm)
    return v


def _do_row(lbuf, gbuf, obuf, off, perms, tiny):
    x = [
        lbuf[pl.ds(off + j * _LANES, _LANES)] + gbuf[pl.ds(off + j * _LANES, _LANES)]
        for j in range(_VPR)
    ]
    e = [jnp.exp(xj) for xj in x]
    s = e[0] + e[1] + e[2] + e[3]
    r = 1.0 / _lane_all_sum(s, perms)
    p = [ej * r for ej in e]
    khot = list(p)
    for _ in range(_K - 1):
        w = [pj * jnp.maximum(1.0 - pj, tiny) for pj in p]
        s = (w[0] + w[1]) + (w[2] + w[3])
        r = 1.0 / _lane_all_sum(s, perms)
        p = [wj * r for wj in w]
        khot = [kj + pj for kj, pj in zip(khot, p)]
    for j in range(_VPR):
        obuf[pl.ds(off + j * _LANES, _LANES)] = khot[j]


def _sc_kernel_body(l_hbm, g_hbm, o_hbm, lbufs, gbufs, obufs, lsems, gsems, osems):
    info = plsc.get_sparse_core_info()
    nc = info.num_cores
    nw = nc * info.num_subcores
    wid = lax.axis_index("s") * nc + lax.axis_index("c")

    n_total = l_hbm.shape[0] // _M
    rows_per_w = n_total // nw
    n_chunks = rows_per_w // _R
    w_base = wid * rows_per_w * _M
    tiny = jnp.float32(jnp.finfo(jnp.float32).tiny)
    lane = lax.iota(jnp.int32, _LANES)
    perms = [lane ^ sh for sh in (1, 2, 4, 8)]

    def start_in(ci, b):
        base = w_base + ci * _CS
        pltpu.make_async_copy(l_hbm.at[pl.ds(base, _CS)], lbufs[b], lsems[b]).start()
        pltpu.make_async_copy(g_hbm.at[pl.ds(base, _CS)], gbufs[b], gsems[b]).start()

    start_in(0, 0)
    start_in(1, 1)

    def pair_body(i, _):
        for b in range(2):
            ci = 2 * i + b
            base = w_base + ci * _CS
            pltpu.make_async_copy(
                l_hbm.at[pl.ds(base, _CS)], lbufs[b], lsems[b]
            ).wait()
            pltpu.make_async_copy(
                g_hbm.at[pl.ds(base, _CS)], gbufs[b], gsems[b]
            ).wait()

            @pl.when(ci >= 2)
            def _():
                pltpu.make_async_copy(
                    obufs[b], o_hbm.at[pl.ds(base - 2 * _CS, _CS)], osems[b]
                ).wait()

            def row_body(ri, _):
                _do_row(lbufs[b], gbufs[b], obufs[b], ri * _M, perms, tiny)
                return 0

            lax.fori_loop(0, _R, row_body, 0)

            pltpu.make_async_copy(
                obufs[b], o_hbm.at[pl.ds(base, _CS)], osems[b]
            ).start()

            @pl.when(ci + 2 < n_chunks)
            def _():
                start_in(ci + 2, b)

        return 0

    lax.fori_loop(0, n_chunks // 2, pair_body, 0)

    for b in range(2):
        ci = n_chunks - 2 + b
        pltpu.make_async_copy(
            obufs[b], o_hbm.at[pl.ds(w_base + ci * _CS, _CS)], osems[b]
        ).wait()


def kernel(logits, gumbel):
    n, m = logits.shape
    mesh = plsc.VectorSubcoreMesh(core_axis_name="c", subcore_axis_name="s")
    buf = lambda: pltpu.VMEM((_CS,), jnp.float32)
    run = functools.partial(
        pl.kernel,
        mesh=mesh,
        out_type=jax.ShapeDtypeStruct((n * m,), jnp.float32),
        scratch_types=[
            [buf(), buf()],
            [buf(), buf()],
            [buf(), buf()],
            [pltpu.SemaphoreType.DMA, pltpu.SemaphoreType.DMA],
            [pltpu.SemaphoreType.DMA, pltpu.SemaphoreType.DMA],
            [pltpu.SemaphoreType.DMA, pltpu.SemaphoreType.DMA],
        ],
    )(_sc_kernel_body)
    out = run(logits.reshape(-1), gumbel.reshape(-1))
    return out.reshape(n, m)
